# Initial kernel scaffold; baseline (speedup 1.0000x reference)
import jax
import jax.numpy as jnp
from jax.experimental import pallas as pl


def kernel(inputs_id, embed_tokens_weight):
    raise NotImplementedError("write your pallas kernel here")


# speedup vs baseline: 1.5058x; 1.5058x over previous
import functools

import jax
import jax.numpy as jnp
from jax import lax
from jax.experimental import pallas as pl
from jax.experimental.pallas import tpu as pltpu
from jax.experimental.pallas import tpu_sc as plsc

NC = 2
NS = 16
NW = NC * NS


@functools.partial(jax.jit, static_argnums=(2, 3))
def _sc_gather(ids, table, n_per_w, chunk):
    N = ids.shape[0]
    D = table.shape[1]
    n_chunks = n_per_w // chunk
    mesh = plsc.VectorSubcoreMesh(
        core_axis_name="c", subcore_axis_name="s",
        num_cores=NC, num_subcores=NS)

    @functools.partial(
        pl.kernel,
        out_type=jax.ShapeDtypeStruct((N, D), jnp.float32),
        mesh=mesh,
        scratch_types=[
            pltpu.VMEM((n_per_w,), jnp.int32),
            pltpu.VMEM((chunk, D), jnp.float32),
            pltpu.SemaphoreType.DMA,
        ],
    )
    def k(idx_hbm, table_hbm, out_hbm, idx_v, rows_v, sem):
        wid = lax.axis_index("s") * NC + lax.axis_index("c")
        base = wid * n_per_w
        pltpu.sync_copy(idx_hbm.at[pl.ds(base, n_per_w)], idx_v)
        for c in range(n_chunks):
            pltpu.async_copy(
                table_hbm.at[idx_v.at[pl.ds(c * chunk, chunk)]],
                rows_v, sem).wait()
            pltpu.sync_copy(rows_v, out_hbm.at[pl.ds(base + c * chunk, chunk)])

    return k(ids, table)


def kernel(inputs_id, embed_tokens_weight):
    B, S = inputs_id.shape
    V, D = embed_tokens_weight.shape
    N = B * S
    ids = inputs_id.reshape(N).astype(jnp.int32)
    out = _sc_gather(ids, embed_tokens_weight, N // NW, 64)
    return out.reshape(B, S, D)


# speedup vs baseline: 1.5324x; 1.0177x over previous
import functools

import jax
import jax.numpy as jnp
from jax import lax
from jax.experimental import pallas as pl
from jax.experimental.pallas import tpu as pltpu
from jax.experimental.pallas import tpu_sc as plsc

NC = 2
NS = 16
NW = NC * NS


@functools.partial(jax.jit, static_argnums=(2, 3, 4))
def _sc_gather(ids, table, n_per_w, chunk, nbuf):
    N = ids.shape[0]
    D = table.shape[1]
    n_chunks = n_per_w // chunk
    mesh = plsc.VectorSubcoreMesh(
        core_axis_name="c", subcore_axis_name="s",
        num_cores=NC, num_subcores=NS)

    @functools.partial(
        pl.kernel,
        out_type=jax.ShapeDtypeStruct((N, D), jnp.float32),
        mesh=mesh,
        scratch_types=[
            pltpu.VMEM((n_per_w,), jnp.int32),
            pltpu.VMEM((nbuf, chunk, D), jnp.float32),
            [pltpu.SemaphoreType.DMA] * nbuf,
            [pltpu.SemaphoreType.DMA] * nbuf,
        ],
    )
    def k(idx_hbm, table_hbm, out_hbm, idx_v, rows_v, gsems, ssems):
        wid = lax.axis_index("s") * NC + lax.axis_index("c")
        base = wid * n_per_w

        pltpu.sync_copy(idx_hbm.at[pl.ds(base, n_per_w)], idx_v)

        def gather(c):
            b = c % nbuf
            return pltpu.async_copy(
                table_hbm.at[idx_v.at[pl.ds(c * chunk, chunk)]],
                rows_v.at[b], gsems[b])

        def scatter(c):
            b = c % nbuf
            return pltpu.async_copy(
                rows_v.at[b], out_hbm.at[pl.ds(base + c * chunk, chunk)],
                ssems[b])

        gd = [None] * n_chunks
        sd = [None] * n_chunks
        gd[0] = gather(0)
        for c in range(n_chunks):
            nxt = c + 1
            if nxt < n_chunks:
                if nxt >= nbuf:
                    sd[nxt - nbuf].wait()
                gd[nxt] = gather(nxt)
            gd[c].wait()
            sd[c] = scatter(c)
        for c in range(max(0, n_chunks - nbuf), n_chunks):
            sd[c].wait()

    return k(ids, table)


def kernel(inputs_id, embed_tokens_weight):
    B, S = inputs_id.shape
    V, D = embed_tokens_weight.shape
    N = B * S
    ids = inputs_id.reshape(N).astype(jnp.int32)
    out = _sc_gather(ids, embed_tokens_weight, N // NW, 32, 3)
    return out.reshape(B, S, D)
